# Initial kernel scaffold; baseline (speedup 1.0000x reference)
import jax
import jax.numpy as jnp
from jax.experimental import pallas as pl


def kernel(x, pos, batch, y, edge_index, W_down, b_down, gamma, beta):
    raise NotImplementedError("write your pallas kernel here")


# speedup vs baseline: 1.4386x; 1.4386x over previous
import jax
import jax.numpy as jnp
import numpy as np
from jax.experimental import pallas as pl

N = 10000
E = 320000
D_IN = 128
D_OUT = 128
NPTS = 5000
EPS = 1e-5


def _down_body(x_ref, w_ref, b_ref, g_ref, be_ref, o_ref):
    h = jnp.dot(x_ref[:], w_ref[:].T, preferred_element_type=jnp.float32)
    h = h + b_ref[:]
    mean = jnp.mean(h, axis=0, keepdims=True)
    var = jnp.mean((h - mean) ** 2, axis=0, keepdims=True)
    h = (h - mean) * jax.lax.rsqrt(var + EPS) * g_ref[:] + be_ref[:]
    o_ref[:] = jnp.maximum(h, 0.0)


def _fps(pos, npoints):
    n = pos.shape[0]
    start = jnp.int32(0)
    dists = jnp.full((n,), jnp.inf, dtype=pos.dtype)
    sel0 = jnp.zeros((npoints,), dtype=jnp.int32).at[0].set(start)

    def body(i, state):
        d_min, sel, last = state
        d = jnp.sum((pos - pos[last]) ** 2, axis=1)
        d_min = jnp.minimum(d_min, d)
        nxt = jnp.argmax(d_min).astype(jnp.int32)
        sel = sel.at[i].set(nxt)
        return (d_min, sel, nxt)

    _, sel, _ = jax.lax.fori_loop(1, npoints, body, (dists, sel0, start))
    return sel


def kernel(x, pos, batch, y, edge_index, W_down, b_down, gamma, beta):
    h = pl.pallas_call(
        _down_body,
        out_shape=jax.ShapeDtypeStruct((N, D_OUT), jnp.float32),
    )(x, W_down, b_down.reshape(1, D_OUT), gamma.reshape(1, D_OUT),
      beta.reshape(1, D_OUT))

    self_idx = jnp.arange(N, dtype=edge_index.dtype)
    row = jnp.concatenate([edge_index[0], self_idx])
    col = jnp.concatenate([edge_index[1], self_idx])
    h_pooled = jax.ops.segment_max(h[row], col, num_segments=N)

    idx = jnp.sort(_fps(pos, NPTS))
    return h_pooled[idx], pos[idx], batch[idx], y[idx]


# speedup vs baseline: 8.1681x; 5.6777x over previous
import jax
import jax.numpy as jnp
import numpy as np
from jax.experimental import pallas as pl

N = 10000
E = 320000
D_IN = 128
D_OUT = 128
NPTS = 5000
EPS = 1e-5
ROWS, COLS = 80, 128
NPAD = ROWS * COLS


def _fps_body(px_ref, py_ref, pz_ref, mask_ref):
    X = px_ref[:]
    Y = py_ref[:]
    Z = pz_ref[:]
    ridx = jax.lax.broadcasted_iota(jnp.int32, (ROWS, COLS), 0)
    cidx = jax.lax.broadcasted_iota(jnp.int32, (ROWS, COLS), 1)
    flat = ridx * COLS + cidx
    valid = flat < N
    d_min0 = jnp.where(valid, jnp.inf, -jnp.inf)
    sel0 = (flat == 0).astype(jnp.int32)
    s0 = sel0 > 0
    lx0 = jnp.sum(jnp.where(s0, X, 0.0))
    ly0 = jnp.sum(jnp.where(s0, Y, 0.0))
    lz0 = jnp.sum(jnp.where(s0, Z, 0.0))

    def body(i, st):
        d_min, mask, lx, ly, lz = st
        dx = X - lx
        dy = Y - ly
        dz = Z - lz
        d = dx * dx + dy * dy + dz * dz
        d_min = jnp.minimum(d_min, d)
        m = jnp.max(d_min)
        cand = jnp.where(d_min == m, flat, jnp.int32(2**30))
        nxt = jnp.min(cand)
        sel = flat == nxt
        mask = mask | sel.astype(jnp.int32)
        lx = jnp.sum(jnp.where(sel, X, 0.0))
        ly = jnp.sum(jnp.where(sel, Y, 0.0))
        lz = jnp.sum(jnp.where(sel, Z, 0.0))
        return d_min, mask, lx, ly, lz

    _, mask, _, _, _ = jax.lax.fori_loop(
        1, NPTS, body, (d_min0, sel0, lx0, ly0, lz0))
    mask_ref[:] = mask


def _fps_mask(pos):
    pad = jnp.zeros((NPAD - N,), jnp.float32)
    px = jnp.concatenate([pos[:, 0], pad]).reshape(ROWS, COLS)
    py = jnp.concatenate([pos[:, 1], pad]).reshape(ROWS, COLS)
    pz = jnp.concatenate([pos[:, 2], pad]).reshape(ROWS, COLS)
    return pl.pallas_call(
        _fps_body,
        out_shape=jax.ShapeDtypeStruct((ROWS, COLS), jnp.int32),
    )(px, py, pz)


def _down_body(x_ref, w_ref, b_ref, g_ref, be_ref, o_ref):
    h = jnp.dot(x_ref[:], w_ref[:].T, preferred_element_type=jnp.float32)
    h = h + b_ref[:]
    mean = jnp.mean(h, axis=0, keepdims=True)
    var = jnp.mean((h - mean) ** 2, axis=0, keepdims=True)
    h = (h - mean) * jax.lax.rsqrt(var + EPS) * g_ref[:] + be_ref[:]
    o_ref[:] = jnp.maximum(h, 0.0)


def kernel(x, pos, batch, y, edge_index, W_down, b_down, gamma, beta):
    h = pl.pallas_call(
        _down_body,
        out_shape=jax.ShapeDtypeStruct((N, D_OUT), jnp.float32),
    )(x, W_down, b_down.reshape(1, D_OUT), gamma.reshape(1, D_OUT),
      beta.reshape(1, D_OUT))

    self_idx = jnp.arange(N, dtype=edge_index.dtype)
    row = jnp.concatenate([edge_index[0], self_idx])
    col = jnp.concatenate([edge_index[1], self_idx])
    h_pooled = jax.ops.segment_max(h[row], col, num_segments=N)

    mask = _fps_mask(pos)
    idx = jnp.nonzero(mask.reshape(-1), size=NPTS, fill_value=0)[0].astype(jnp.int32)
    return h_pooled[idx], pos[idx], batch[idx], y[idx]


# speedup vs baseline: 11.0952x; 1.3584x over previous
import jax
import jax.numpy as jnp
from jax import lax
from jax.experimental import pallas as pl
from jax.experimental.pallas import tpu as pltpu
from jax.experimental.pallas import tpu_sc as plsc

N = 10000
E = 320000
D_IN = 128
D_OUT = 128
NPTS = 5000
EPS = 1e-5
ROWS, COLS = 80, 128
NPAD = ROWS * COLS

NC, NS = 2, 16
NW = NC * NS
RPW = NPAD // NW
TRASH = RPW
CHUNK = 3200
NCHUNK = E // CHUNK
GRPS = CHUNK // 16
HCAP = 16384
NG2 = RPW // 16


def _down_body(x_ref, w_ref, b_ref, g_ref, be_ref, o_ref):
    h = jnp.dot(x_ref[:], w_ref[:].T, preferred_element_type=jnp.float32)
    h = h + b_ref[:]
    mean = jnp.mean(h, axis=0, keepdims=True)
    var = jnp.mean((h - mean) ** 2, axis=0, keepdims=True)
    h = (h - mean) * jax.lax.rsqrt(var + EPS) * g_ref[:] + be_ref[:]
    o_ref[pl.ds(0, N), :] = jnp.maximum(h, 0.0)
    o_ref[pl.ds(N, NPAD - N), :] = jnp.zeros((NPAD - N, D_OUT), jnp.float32)


def _down(x, W_down, b_down, gamma, beta):
    return pl.pallas_call(
        _down_body,
        out_shape=jax.ShapeDtypeStruct((NPAD, D_OUT), jnp.float32),
    )(x, W_down, b_down.reshape(1, D_OUT), gamma.reshape(1, D_OUT),
      beta.reshape(1, D_OUT))


def _fps_body(px_ref, py_ref, pz_ref, mask_ref, rank_ref):
    X = px_ref[:]
    Y = py_ref[:]
    Z = pz_ref[:]
    ridx = jax.lax.broadcasted_iota(jnp.int32, (ROWS, COLS), 0)
    cidx = jax.lax.broadcasted_iota(jnp.int32, (ROWS, COLS), 1)
    flat = ridx * COLS + cidx
    valid = flat < N
    d_min0 = jnp.where(valid, jnp.inf, -jnp.inf)
    sel0 = (flat == 0).astype(jnp.int32)
    s0 = sel0 > 0
    lx0 = jnp.sum(jnp.where(s0, X, 0.0))
    ly0 = jnp.sum(jnp.where(s0, Y, 0.0))
    lz0 = jnp.sum(jnp.where(s0, Z, 0.0))

    def body(i, st):
        d_min, mask, lx, ly, lz = st
        dx = X - lx
        dy = Y - ly
        dz = Z - lz
        d = dx * dx + dy * dy + dz * dz
        d_min = jnp.minimum(d_min, d)
        m = jnp.max(d_min)
        cand = jnp.where(d_min == m, flat, jnp.int32(2**30))
        nxt = jnp.min(cand)
        sel = flat == nxt
        mask = mask | sel.astype(jnp.int32)
        lx = jnp.sum(jnp.where(sel, X, 0.0))
        ly = jnp.sum(jnp.where(sel, Y, 0.0))
        lz = jnp.sum(jnp.where(sel, Z, 0.0))
        return d_min, mask, lx, ly, lz

    _, mask, _, _, _ = jax.lax.fori_loop(
        1, NPTS, body, (d_min0, sel0, lx0, ly0, lz0))
    mask_ref[:] = mask

    maskf = mask.astype(jnp.float32)
    ci = jax.lax.broadcasted_iota(jnp.int32, (COLS, COLS), 0)
    cj = jax.lax.broadcasted_iota(jnp.int32, (COLS, COLS), 1)
    U = (ci <= cj).astype(jnp.float32)
    incl = jnp.dot(maskf, U, preferred_element_type=jnp.float32)
    ones = jnp.ones((COLS, COLS), jnp.float32)
    rowtot = jnp.dot(maskf, ones, preferred_element_type=jnp.float32)
    ri = jax.lax.broadcasted_iota(jnp.int32, (ROWS, ROWS), 0)
    rj = jax.lax.broadcasted_iota(jnp.int32, (ROWS, ROWS), 1)
    Ls = (rj < ri).astype(jnp.float32)
    prevrows = jnp.dot(Ls, rowtot, preferred_element_type=jnp.float32)
    rank = prevrows + incl - maskf
    rank_ref[:] = rank.astype(jnp.int32)


def _fps_mask_rank(px, py, pz):
    return pl.pallas_call(
        _fps_body,
        out_shape=(jax.ShapeDtypeStruct((ROWS, COLS), jnp.int32),
                   jax.ShapeDtypeStruct((ROWS, COLS), jnp.int32)),
    )(px, py, pz)


def _k1_body(h_hbm, row_hbm, col_hbm, out_hbm,
             acc, colbuf0, colbuf1, rowbuf0, rowbuf1,
             hitrow, hitcol, gbuf0, gbuf1,
             csem0, csem1, gsem0, gsem1):
    wid = lax.axis_index("s") * NC + lax.axis_index("c")
    lo = wid * RPW
    pltpu.sync_copy(h_hbm.at[pl.ds(lo, RPW)], acc.at[pl.ds(0, RPW)])

    def issue_chunk(c, colbuf, rowbuf, sem):
        pltpu.async_copy(col_hbm.at[pl.ds(c * CHUNK, CHUNK)], colbuf, sem)
        pltpu.async_copy(row_hbm.at[pl.ds(c * CHUNK, CHUNK)], rowbuf, sem)

    def wait_chunk(c, colbuf, rowbuf, sem):
        pltpu.make_async_copy(
            col_hbm.at[pl.ds(c * CHUNK, CHUNK)], colbuf, sem).wait()
        pltpu.make_async_copy(
            row_hbm.at[pl.ds(c * CHUNK, CHUNK)], rowbuf, sem).wait()

    def scan_chunk(colbuf, rowbuf, off):
        def g_body(g, off):
            colv = colbuf[pl.ds(g * 16, 16)]
            a = colv - lo
            b = (lo + RPW - 1) - colv
            inr = 1 - lax.shift_right_logical(a | b, 31)
            cnt = inr[0]
            for k in range(1, 16):
                cnt = cnt + inr[k]

            def do_hit(o):
                rowv = rowbuf[pl.ds(g * 16, 16)]
                cloc = colv - lo
                for k in range(16):
                    hitcol[pl.ds(o, 16)] = jnp.full((16,), cloc[k], jnp.int32)
                    hitrow[pl.ds(o, 16)] = jnp.full((16,), rowv[k], jnp.int32)
                    o = o + inr[k]
                return o

            return lax.cond(cnt > 0, do_hit, lambda o: o, off)

        return lax.fori_loop(0, GRPS, g_body, off)

    issue_chunk(0, colbuf0, rowbuf0, csem0)

    def a_body(c, off):
        def even(off):
            @pl.when(c + 1 < NCHUNK)
            def _():
                issue_chunk(c + 1, colbuf1, rowbuf1, csem1)
            wait_chunk(c, colbuf0, rowbuf0, csem0)
            return scan_chunk(colbuf0, rowbuf0, off)

        def odd(off):
            @pl.when(c + 1 < NCHUNK)
            def _():
                issue_chunk(c + 1, colbuf0, rowbuf0, csem0)
            wait_chunk(c, colbuf1, rowbuf1, csem1)
            return scan_chunk(colbuf1, rowbuf1, off)

        return lax.cond(c % 2 == 0, even, odd, off)

    nh = lax.fori_loop(0, NCHUNK, a_body, jnp.int32(0))

    hitcol[pl.ds(nh, 16)] = jnp.full((16,), TRASH, jnp.int32)
    hitrow[pl.ds(nh, 16)] = jnp.zeros((16,), jnp.int32)
    ng = (nh + 15) // 16

    def issue_g(g, buf, sem):
        pltpu.async_copy(h_hbm.at[hitrow.at[pl.ds(g * 16, 16)]], buf, sem)

    def wait_g(g, buf, sem):
        pltpu.make_async_copy(
            h_hbm.at[hitrow.at[pl.ds(g * 16, 16)]], buf, sem).wait()

    def accum(g, buf):
        hc = hitcol[pl.ds(g * 16, 16)]
        for k in range(16):
            cl = hc[k]
            for j in range(8):
                sl = pl.ds(j * 16, 16)
                acc[cl, sl] = jnp.maximum(acc[cl, sl], buf[k, sl])

    @pl.when(ng > 0)
    def _():
        issue_g(0, gbuf0, gsem0)

    def b_body(g, _):
        def even(_):
            @pl.when(g + 1 < ng)
            def _():
                issue_g(g + 1, gbuf1, gsem1)
            wait_g(g, gbuf0, gsem0)
            accum(g, gbuf0)
            return 0

        def odd(_):
            @pl.when(g + 1 < ng)
            def _():
                issue_g(g + 1, gbuf0, gsem0)
            wait_g(g, gbuf1, gsem1)
            accum(g, gbuf1)
            return 0

        return lax.cond(g % 2 == 0, even, odd, 0)

    lax.fori_loop(0, ng, b_body, 0)

    pltpu.sync_copy(acc.at[pl.ds(0, RPW)], out_hbm.at[pl.ds(lo, RPW)])


_k1_call = pl.kernel(
    _k1_body,
    out_type=jax.ShapeDtypeStruct((NPAD, D_OUT), jnp.float32),
    mesh=plsc.VectorSubcoreMesh(core_axis_name="c", subcore_axis_name="s"),
    scratch_types=[
        pltpu.VMEM((RPW + 1, D_OUT), jnp.float32),
        pltpu.VMEM((CHUNK,), jnp.int32),
        pltpu.VMEM((CHUNK,), jnp.int32),
        pltpu.VMEM((CHUNK,), jnp.int32),
        pltpu.VMEM((CHUNK,), jnp.int32),
        pltpu.VMEM((HCAP,), jnp.int32),
        pltpu.VMEM((HCAP,), jnp.int32),
        pltpu.VMEM((16, D_OUT), jnp.float32),
        pltpu.VMEM((16, D_OUT), jnp.float32),
        pltpu.SemaphoreType.DMA,
        pltpu.SemaphoreType.DMA,
        pltpu.SemaphoreType.DMA,
        pltpu.SemaphoreType.DMA,
    ],
)


def _k2_body(hp_hbm, mask_hbm, rank_hbm, px_hbm, py_hbm, pz_hbm,
             ypad_hbm, bpad_hbm,
             oh_hbm, opos_hbm, ob_hbm, oy_hbm,
             maskb, rankb, pxb, pyb, pzb, yb, bb,
             slotb, pix0, pix1, pix2, rowstage,
             wsem, esem):
    wid = lax.axis_index("s") * NC + lax.axis_index("c")
    lo = wid * RPW
    pltpu.sync_copy(mask_hbm.at[pl.ds(lo, RPW)], maskb)
    pltpu.sync_copy(rank_hbm.at[pl.ds(lo, RPW)], rankb)
    pltpu.sync_copy(px_hbm.at[pl.ds(lo, RPW)], pxb)
    pltpu.sync_copy(py_hbm.at[pl.ds(lo, RPW)], pyb)
    pltpu.sync_copy(pz_hbm.at[pl.ds(lo, RPW)], pzb)
    pltpu.sync_copy(ypad_hbm.at[pl.ds(lo, RPW)], yb)
    pltpu.sync_copy(bpad_hbm.at[pl.ds(lo, RPW)], bb)

    for gi in range(NG2):
        mv = maskb[pl.ds(gi * 16, 16)]
        rv = rankb[pl.ds(gi * 16, 16)]
        slots = rv * mv + jnp.int32(NPTS) * (1 - mv)
        slotb[gi, :] = slots
        pix0[gi, :] = slots * 3
        pix1[gi, :] = slots * 3 + 1
        pix2[gi, :] = slots * 3 + 2

    for gi in range(NG2):
        pltpu.async_copy(pxb.at[pl.ds(gi * 16, 16)],
                         opos_hbm.at[pix0.at[gi]], esem)
        pltpu.async_copy(pyb.at[pl.ds(gi * 16, 16)],
                         opos_hbm.at[pix1.at[gi]], esem)
        pltpu.async_copy(pzb.at[pl.ds(gi * 16, 16)],
                         opos_hbm.at[pix2.at[gi]], esem)
        pltpu.async_copy(yb.at[pl.ds(gi * 16, 16)],
                         oy_hbm.at[slotb.at[gi]], esem)
        pltpu.async_copy(bb.at[pl.ds(gi * 16, 16)],
                         ob_hbm.at[slotb.at[gi]], esem)

    for gi in range(NG2):
        par = gi % 2
        if gi >= 2:
            pltpu.make_async_copy(
                rowstage.at[par], oh_hbm.at[slotb.at[gi - 2]], wsem).wait()
        pltpu.sync_copy(hp_hbm.at[pl.ds(lo + gi * 16, 16)], rowstage.at[par])
        pltpu.async_copy(rowstage.at[par], oh_hbm.at[slotb.at[gi]], wsem)

    for gi in (NG2 - 2, NG2 - 1):
        pltpu.make_async_copy(
            rowstage.at[gi % 2], oh_hbm.at[slotb.at[gi]], wsem).wait()

    for gi in range(NG2):
        pltpu.make_async_copy(pxb.at[pl.ds(gi * 16, 16)],
                              opos_hbm.at[pix0.at[gi]], esem).wait()
        pltpu.make_async_copy(pyb.at[pl.ds(gi * 16, 16)],
                              opos_hbm.at[pix1.at[gi]], esem).wait()
        pltpu.make_async_copy(pzb.at[pl.ds(gi * 16, 16)],
                              opos_hbm.at[pix2.at[gi]], esem).wait()
        pltpu.make_async_copy(yb.at[pl.ds(gi * 16, 16)],
                              oy_hbm.at[slotb.at[gi]], esem).wait()
        pltpu.make_async_copy(bb.at[pl.ds(gi * 16, 16)],
                              ob_hbm.at[slotb.at[gi]], esem).wait()


_k2_call = pl.kernel(
    _k2_body,
    out_type=(
        jax.ShapeDtypeStruct((NPTS + 1, D_OUT), jnp.float32),
        jax.ShapeDtypeStruct((3 * NPTS + 3,), jnp.float32),
        jax.ShapeDtypeStruct((NPTS + 1,), jnp.int32),
        jax.ShapeDtypeStruct((NPTS + 1,), jnp.int32),
    ),
    mesh=plsc.VectorSubcoreMesh(core_axis_name="c", subcore_axis_name="s"),
    scratch_types=[
        pltpu.VMEM((RPW,), jnp.int32),
        pltpu.VMEM((RPW,), jnp.int32),
        pltpu.VMEM((RPW,), jnp.float32),
        pltpu.VMEM((RPW,), jnp.float32),
        pltpu.VMEM((RPW,), jnp.float32),
        pltpu.VMEM((RPW,), jnp.int32),
        pltpu.VMEM((RPW,), jnp.int32),
        pltpu.VMEM((NG2, 16), jnp.int32),
        pltpu.VMEM((NG2, 16), jnp.int32),
        pltpu.VMEM((NG2, 16), jnp.int32),
        pltpu.VMEM((NG2, 16), jnp.int32),
        pltpu.VMEM((2, 16, D_OUT), jnp.float32),
        pltpu.SemaphoreType.DMA,
        pltpu.SemaphoreType.DMA,
    ],
)


def kernel(x, pos, batch, y, edge_index, W_down, b_down, gamma, beta):
    pad = jnp.zeros((NPAD - N,), jnp.float32)
    px = jnp.concatenate([pos[:, 0], pad]).reshape(ROWS, COLS)
    py = jnp.concatenate([pos[:, 1], pad]).reshape(ROWS, COLS)
    pz = jnp.concatenate([pos[:, 2], pad]).reshape(ROWS, COLS)
    ipad = jnp.zeros((NPAD - N,), jnp.int32)
    y_pad = jnp.concatenate([y, ipad])
    b_pad = jnp.concatenate([batch, ipad])

    h = _down(x, W_down, b_down, gamma, beta)
    maskm, rankm = _fps_mask_rank(px, py, pz)
    hp = _k1_call(h, edge_index[0], edge_index[1])
    idx = jnp.nonzero(maskm.reshape(-1), size=NPTS, fill_value=0)[0].astype(jnp.int32)
    return hp[idx], pos[idx], batch[idx], y[idx]
